# baseline (speedup 1.0000x reference)
import functools

import jax
import jax.numpy as jnp
from jax import lax
from jax.experimental import pallas as pl
from jax.experimental.pallas import tpu as pltpu
from jax.experimental.pallas import tpu_sc as plsc

_NB = 128


@functools.cache
def _make_sc_gather(n_rows: int, d: int, batch: int):
    info = plsc.get_sparse_core_info()
    nc, ns = info.num_cores, info.num_subcores
    nw = nc * ns
    b_per_w = batch // nw
    mesh = plsc.VectorSubcoreMesh(core_axis_name="c", subcore_axis_name="s")

    @functools.partial(
        pl.kernel,
        mesh=mesh,
        out_type=jax.ShapeDtypeStruct((batch, d), jnp.float32),
        scratch_types=[
            pltpu.VMEM((b_per_w,), jnp.int32),
            pltpu.VMEM((b_per_w, d), jnp.float32),
            pltpu.SemaphoreType.DMA,
        ],
    )
    def gather_k(table_hbm, idx_hbm, out_hbm, idx_v, rows_v, sem):
        wid = lax.axis_index("s") * nc + lax.axis_index("c")
        base = wid * b_per_w
        pltpu.sync_copy(idx_hbm.at[pl.ds(base, b_per_w)], idx_v)
        pltpu.async_copy(table_hbm.at[idx_v], rows_v, sem).wait()
        pltpu.sync_copy(rows_v, out_hbm.at[pl.ds(base, b_per_w)])

    return gather_k


@functools.cache
def _make_assemble(batch: int, n_win: int, d_lc: int, d_out: int):
    n_ch = batch // _NB
    d_em = d_out - d_lc

    def body(lc_ref, gc_ref, out_ref):
        out_ref[:, :, :d_lc] = lc_ref[...]
        gc = gc_ref[:, :d_em]
        out_ref[:, :, d_lc:] = jnp.broadcast_to(gc[:, None, :], (_NB, n_win, d_em))

    return pl.pallas_call(
        body,
        grid=(n_ch,),
        in_specs=[
            pl.BlockSpec((_NB, n_win, d_lc), lambda c: (c, 0, 0)),
            pl.BlockSpec((_NB, 128), lambda c: (c, 0)),
        ],
        out_specs=pl.BlockSpec((_NB, n_win, d_out), lambda c: (c, 0, 0)),
        out_shape=jax.ShapeDtypeStruct((batch, n_win, d_out), jnp.float32),
    )


def kernel(lc, ids, W, b):
    batch, n_win, d_lc = lc.shape
    n_embed = W.shape[0]
    table = jnp.zeros((W.shape[1], 128), jnp.float32)
    table = table.at[:, :n_embed].set(W.T + b[None, :])
    idx = ids.astype(jnp.int32)

    gc = _make_sc_gather(table.shape[0], 128, batch)(table, idx)

    out = _make_assemble(batch, n_win, d_lc, d_lc + n_embed)(lc, gc)
    return out


# speedup vs baseline: 1.0060x; 1.0060x over previous
import functools

import jax
import jax.numpy as jnp
from jax import lax
from jax.experimental import pallas as pl
from jax.experimental.pallas import tpu as pltpu
from jax.experimental.pallas import tpu_sc as plsc

_NB = 256


@functools.cache
def _make_sc_gather(n_rows: int, d: int, batch: int):
    info = plsc.get_sparse_core_info()
    nc, ns = info.num_cores, info.num_subcores
    nw = nc * ns
    b_per_w = batch // nw
    mesh = plsc.VectorSubcoreMesh(core_axis_name="c", subcore_axis_name="s")

    @functools.partial(
        pl.kernel,
        mesh=mesh,
        out_type=jax.ShapeDtypeStruct((batch, d), jnp.float32),
        scratch_types=[
            pltpu.VMEM((b_per_w,), jnp.int32),
            pltpu.VMEM((b_per_w, d), jnp.float32),
            pltpu.SemaphoreType.DMA,
        ],
    )
    def gather_k(table_hbm, idx_hbm, out_hbm, idx_v, rows_v, sem):
        wid = lax.axis_index("s") * nc + lax.axis_index("c")
        base = wid * b_per_w
        pltpu.sync_copy(idx_hbm.at[pl.ds(base, b_per_w)], idx_v)
        pltpu.async_copy(table_hbm.at[idx_v], rows_v, sem).wait()
        pltpu.sync_copy(rows_v, out_hbm.at[pl.ds(base, b_per_w)])

    return gather_k


@functools.cache
def _make_assemble(batch: int, n_win: int, d_lc: int, d_out: int):
    n_ch = batch // _NB
    d_em = d_out - d_lc

    def body(lc_ref, gc_ref, out_ref):
        out_ref[:, :, :d_lc] = lc_ref[...]
        gc = gc_ref[:, :d_em]
        out_ref[:, :, d_lc:] = jnp.broadcast_to(gc[:, None, :], (_NB, n_win, d_em))

    return pl.pallas_call(
        body,
        grid=(n_ch,),
        in_specs=[
            pl.BlockSpec((_NB, n_win, d_lc), lambda c: (c, 0, 0)),
            pl.BlockSpec((_NB, 128), lambda c: (c, 0)),
        ],
        out_specs=pl.BlockSpec((_NB, n_win, d_out), lambda c: (c, 0, 0)),
        out_shape=jax.ShapeDtypeStruct((batch, n_win, d_out), jnp.float32),
    )


def kernel(lc, ids, W, b):
    batch, n_win, d_lc = lc.shape
    n_embed = W.shape[0]
    table = jnp.zeros((W.shape[1], 128), jnp.float32)
    table = table.at[:, :n_embed].set(W.T + b[None, :])
    idx = ids.astype(jnp.int32)

    gc = _make_sc_gather(table.shape[0], 128, batch)(table, idx)

    out = _make_assemble(batch, n_win, d_lc, d_lc + n_embed)(lc, gc)
    return out


# speedup vs baseline: 1.0118x; 1.0058x over previous
import functools

import jax
import jax.numpy as jnp
from jax import lax
from jax.experimental import pallas as pl
from jax.experimental.pallas import tpu as pltpu
from jax.experimental.pallas import tpu_sc as plsc

_NB = 512


@functools.cache
def _make_sc_gather(n_rows: int, d: int, batch: int):
    info = plsc.get_sparse_core_info()
    nc, ns = info.num_cores, info.num_subcores
    nw = nc * ns
    b_per_w = batch // nw
    mesh = plsc.VectorSubcoreMesh(core_axis_name="c", subcore_axis_name="s")

    @functools.partial(
        pl.kernel,
        mesh=mesh,
        out_type=jax.ShapeDtypeStruct((batch, d), jnp.float32),
        scratch_types=[
            pltpu.VMEM((b_per_w,), jnp.int32),
            pltpu.VMEM((b_per_w, d), jnp.float32),
            pltpu.SemaphoreType.DMA,
        ],
    )
    def gather_k(table_hbm, idx_hbm, out_hbm, idx_v, rows_v, sem):
        wid = lax.axis_index("s") * nc + lax.axis_index("c")
        base = wid * b_per_w
        pltpu.sync_copy(idx_hbm.at[pl.ds(base, b_per_w)], idx_v)
        pltpu.async_copy(table_hbm.at[idx_v], rows_v, sem).wait()
        pltpu.sync_copy(rows_v, out_hbm.at[pl.ds(base, b_per_w)])

    return gather_k


@functools.cache
def _make_assemble(batch: int, n_win: int, d_lc: int, d_out: int):
    n_ch = batch // _NB
    d_em = d_out - d_lc

    def body(lc_ref, gc_ref, out_ref):
        out_ref[:, :, :d_lc] = lc_ref[...]
        gc = gc_ref[:, :d_em]
        out_ref[:, :, d_lc:] = jnp.broadcast_to(gc[:, None, :], (_NB, n_win, d_em))

    return pl.pallas_call(
        body,
        grid=(n_ch,),
        in_specs=[
            pl.BlockSpec((_NB, n_win, d_lc), lambda c: (c, 0, 0)),
            pl.BlockSpec((_NB, 128), lambda c: (c, 0)),
        ],
        out_specs=pl.BlockSpec((_NB, n_win, d_out), lambda c: (c, 0, 0)),
        out_shape=jax.ShapeDtypeStruct((batch, n_win, d_out), jnp.float32),
    )


def kernel(lc, ids, W, b):
    batch, n_win, d_lc = lc.shape
    n_embed = W.shape[0]
    table = jnp.zeros((W.shape[1], 128), jnp.float32)
    table = table.at[:, :n_embed].set(W.T + b[None, :])
    idx = ids.astype(jnp.int32)

    gc = _make_sc_gather(table.shape[0], 128, batch)(table, idx)

    out = _make_assemble(batch, n_win, d_lc, d_lc + n_embed)(lc, gc)
    return out


# speedup vs baseline: 1.0464x; 1.0342x over previous
import functools

import jax
import jax.numpy as jnp
from jax import lax
from jax.experimental import pallas as pl
from jax.experimental.pallas import tpu as pltpu
from jax.experimental.pallas import tpu_sc as plsc

_NB = 128
_NS = 6


@functools.cache
def _make_sc_gather(n_rows: int, d: int, batch: int):
    info = plsc.get_sparse_core_info()
    nc, ns = info.num_cores, info.num_subcores
    nw = nc * ns
    b_per_w = batch // nw
    mesh = plsc.VectorSubcoreMesh(core_axis_name="c", subcore_axis_name="s")

    @functools.partial(
        pl.kernel,
        mesh=mesh,
        out_type=jax.ShapeDtypeStruct((batch, d), jnp.float32),
        scratch_types=[
            pltpu.VMEM((b_per_w,), jnp.int32),
            pltpu.VMEM((b_per_w, d), jnp.float32),
            pltpu.SemaphoreType.DMA,
        ],
    )
    def gather_k(table_hbm, idx_hbm, out_hbm, idx_v, rows_v, sem):
        wid = lax.axis_index("s") * nc + lax.axis_index("c")
        base = wid * b_per_w
        pltpu.sync_copy(idx_hbm.at[pl.ds(base, b_per_w)], idx_v)
        pltpu.async_copy(table_hbm.at[idx_v], rows_v, sem).wait()
        pltpu.sync_copy(rows_v, out_hbm.at[pl.ds(base, b_per_w)])

    return gather_k


@functools.cache
def _make_assemble(batch: int, n_win: int, d_lc: int, d_out: int):
    n_ch = batch // _NB
    d_em = d_out - d_lc

    def body(lc_hbm, gc_hbm, out_hbm, gcb, lbuf, abuf, s_gc, s_lc, s_out):
        def lc_copy(c):
            return pltpu.make_async_copy(
                lc_hbm.at[pl.ds(c * _NB, _NB)], lbuf.at[c % _NS],
                s_lc.at[c % _NS])

        def out_copy(c):
            return pltpu.make_async_copy(
                abuf.at[c % _NS], out_hbm.at[pl.ds(c * _NB, _NB)],
                s_out.at[c % _NS])

        gc_cp = pltpu.make_async_copy(gc_hbm, gcb, s_gc)
        gc_cp.start()
        for c in range(_NS):
            lc_copy(c).start()
        gc_cp.wait()

        for c in range(n_ch):
            slot = c % _NS
            lc_copy(c).wait()
            if c >= _NS:
                out_copy(c - _NS).wait()
            abuf[slot, :, :, :d_lc] = lbuf[slot]
            gc_sl = gcb[pl.ds(c * _NB, _NB), :d_em]
            abuf[slot, :, :, d_lc:] = jnp.broadcast_to(
                gc_sl[:, None, :], (_NB, n_win, d_em))
            out_copy(c).start()
            if c + _NS < n_ch:
                lc_copy(c + _NS).start()

        for c in range(n_ch - _NS, n_ch):
            out_copy(c).wait()

    return pl.pallas_call(
        body,
        in_specs=[
            pl.BlockSpec(memory_space=pl.ANY),
            pl.BlockSpec(memory_space=pl.ANY),
        ],
        out_specs=pl.BlockSpec(memory_space=pl.ANY),
        out_shape=jax.ShapeDtypeStruct((batch, n_win, d_out), jnp.float32),
        scratch_shapes=[
            pltpu.VMEM((batch, 128), jnp.float32),
            pltpu.VMEM((_NS, _NB, n_win, d_lc), jnp.float32),
            pltpu.VMEM((_NS, _NB, n_win, d_out), jnp.float32),
            pltpu.SemaphoreType.DMA,
            pltpu.SemaphoreType.DMA((_NS,)),
            pltpu.SemaphoreType.DMA((_NS,)),
        ],
    )


def kernel(lc, ids, W, b):
    batch, n_win, d_lc = lc.shape
    n_embed = W.shape[0]
    table = jnp.zeros((W.shape[1], 128), jnp.float32)
    table = table.at[:, :n_embed].set(W.T + b[None, :])
    idx = ids.astype(jnp.int32)

    gc = _make_sc_gather(table.shape[0], 128, batch)(table, idx)

    out = _make_assemble(batch, n_win, d_lc, d_lc + n_embed)(lc, gc)
    return out


# speedup vs baseline: 1.0489x; 1.0024x over previous
import functools

import jax
import jax.numpy as jnp
from jax import lax
from jax.experimental import pallas as pl
from jax.experimental.pallas import tpu as pltpu
from jax.experimental.pallas import tpu_sc as plsc

_NB = 128
_NS = 8


@functools.cache
def _make_sc_gather(n_rows: int, d: int, batch: int):
    info = plsc.get_sparse_core_info()
    nc, ns = info.num_cores, info.num_subcores
    nw = nc * ns
    b_per_w = batch // nw
    mesh = plsc.VectorSubcoreMesh(core_axis_name="c", subcore_axis_name="s")

    @functools.partial(
        pl.kernel,
        mesh=mesh,
        out_type=jax.ShapeDtypeStruct((batch, d), jnp.float32),
        scratch_types=[
            pltpu.VMEM((b_per_w,), jnp.int32),
            pltpu.VMEM((b_per_w, d), jnp.float32),
            pltpu.SemaphoreType.DMA,
        ],
    )
    def gather_k(table_hbm, idx_hbm, out_hbm, idx_v, rows_v, sem):
        wid = lax.axis_index("s") * nc + lax.axis_index("c")
        base = wid * b_per_w
        pltpu.sync_copy(idx_hbm.at[pl.ds(base, b_per_w)], idx_v)
        pltpu.async_copy(table_hbm.at[idx_v], rows_v, sem).wait()
        pltpu.sync_copy(rows_v, out_hbm.at[pl.ds(base, b_per_w)])

    return gather_k


@functools.cache
def _make_assemble(batch: int, n_win: int, d_lc: int, d_out: int):
    n_ch = batch // _NB
    d_em = d_out - d_lc

    def body(lc_hbm, gc_hbm, out_hbm, gcb, lbuf, abuf, s_gc, s_lc, s_out):
        def lc_copy(c):
            return pltpu.make_async_copy(
                lc_hbm.at[pl.ds(c * _NB, _NB)], lbuf.at[c % _NS],
                s_lc.at[c % _NS])

        def out_copy(c):
            return pltpu.make_async_copy(
                abuf.at[c % _NS], out_hbm.at[pl.ds(c * _NB, _NB)],
                s_out.at[c % _NS])

        gc_cp = pltpu.make_async_copy(gc_hbm, gcb, s_gc)
        gc_cp.start()
        for c in range(_NS):
            lc_copy(c).start()
        gc_cp.wait()

        for c in range(n_ch):
            slot = c % _NS
            lc_copy(c).wait()
            if c >= _NS:
                out_copy(c - _NS).wait()
            abuf[slot, :, :, :d_lc] = lbuf[slot]
            gc_sl = gcb[pl.ds(c * _NB, _NB), :d_em]
            abuf[slot, :, :, d_lc:] = jnp.broadcast_to(
                gc_sl[:, None, :], (_NB, n_win, d_em))
            out_copy(c).start()
            if c + _NS < n_ch:
                lc_copy(c + _NS).start()

        for c in range(n_ch - _NS, n_ch):
            out_copy(c).wait()

    return pl.pallas_call(
        body,
        in_specs=[
            pl.BlockSpec(memory_space=pl.ANY),
            pl.BlockSpec(memory_space=pl.ANY),
        ],
        out_specs=pl.BlockSpec(memory_space=pl.ANY),
        out_shape=jax.ShapeDtypeStruct((batch, n_win, d_out), jnp.float32),
        scratch_shapes=[
            pltpu.VMEM((batch, 128), jnp.float32),
            pltpu.VMEM((_NS, _NB, n_win, d_lc), jnp.float32),
            pltpu.VMEM((_NS, _NB, n_win, d_out), jnp.float32),
            pltpu.SemaphoreType.DMA,
            pltpu.SemaphoreType.DMA((_NS,)),
            pltpu.SemaphoreType.DMA((_NS,)),
        ],
    )


def kernel(lc, ids, W, b):
    batch, n_win, d_lc = lc.shape
    n_embed = W.shape[0]
    table = jnp.zeros((W.shape[1], 128), jnp.float32)
    table = table.at[:, :n_embed].set(W.T + b[None, :])
    idx = ids.astype(jnp.int32)

    gc = _make_sc_gather(table.shape[0], 128, batch)(table, idx)

    out = _make_assemble(batch, n_win, d_lc, d_lc + n_embed)(lc, gc)
    return out
